# Initial kernel scaffold; baseline (speedup 1.0000x reference)
import jax
import jax.numpy as jnp
from jax.experimental import pallas as pl


def kernel(gu_ids, gu_edge_index, gv_ids, gv_edge_index, emb, Wp1, Wp2, Wc1, Wc2):
    raise NotImplementedError("write your pallas kernel here")


# speedup vs baseline: 16.9896x; 16.9896x over previous
import functools

import jax
import jax.numpy as jnp
from jax import lax
from jax.experimental import pallas as pl
from jax.experimental.pallas import tpu as pltpu
from jax.experimental.pallas import tpu_sc as plsc

N = 10000
E = 320000
V = 50000
D = 128

NC = 2
NS = 16

ROWS_T = 624
EK = 128
ECHUNKS = E // EK
EJ = -(-ECHUNKS // NS)
IK = 80
ICHUNKS = N // IK
IJ = -(-ICHUNKS // NS)
SPN = 10112
RPAD = 20096

_mesh = plsc.VectorSubcoreMesh(core_axis_name="c", subcore_axis_name="s")


def _zero_vec(ref, nwords):
    def body(i, _):
        ref[pl.ds(i * 16, 16)] = jnp.zeros((16,), jnp.float32)
        return 0
    lax.fori_loop(0, nwords // 16, body, 0)


def _tile_node_init(zbuf, shared, sid):
    pltpu.sync_copy(zbuf.at[pl.ds(0, ROWS_T)], shared.at[pl.ds(sid * ROWS_T, ROWS_T)])
    @pl.when(sid == NS - 1)
    def _():
        pltpu.sync_copy(zbuf.at[pl.ds(0, 16)], shared.at[pl.ds(NS * ROWS_T, 16)])


def _tile_node_out(shared, out, sid, base, stage):
    pltpu.sync_copy(shared.at[pl.ds(sid * ROWS_T, ROWS_T)],
                    stage.at[pl.ds(0, ROWS_T)])
    pltpu.sync_copy(stage.at[pl.ds(0, ROWS_T)],
                    out.at[pl.ds(base + sid * ROWS_T, ROWS_T)])
    @pl.when(sid == NS - 1)
    def _():
        pltpu.sync_copy(shared.at[pl.ds(NS * ROWS_T, 16)], stage.at[pl.ds(0, 16)])
        pltpu.sync_copy(stage.at[pl.ds(0, 16)],
                        out.at[pl.ds(base + NS * ROWS_T, 16)])


@functools.partial(
    pl.kernel,
    out_type=(jax.ShapeDtypeStruct((2 * N, D), jnp.float32),
              jax.ShapeDtypeStruct((2 * N,), jnp.float32)),
    mesh=_mesh,
    scratch_types=[
        pltpu.VMEM((IK,), jnp.int32),
        pltpu.VMEM((IK, D), jnp.float32),
        pltpu.VMEM((1, EK), jnp.int32),
        pltpu.VMEM((EK,), jnp.float32),
        pltpu.VMEM((ROWS_T + 16,), jnp.float32),
        pltpu.VMEM_SHARED((SPN,), jnp.float32),
        pltpu.SemaphoreType.DMA,
    ],
)
def _sc_deg_gather(ids2, dst2, emb, h_out, deg_out,
                   idbuf, rowbuf, dstbuf, onesbuf, zbuf, degsp, sem):
    cid = lax.axis_index("c")
    sid = lax.axis_index("s")

    def fill_ones(i, _):
        onesbuf[pl.ds(i * 16, 16)] = jnp.full((16,), 1.0, jnp.float32)
        return 0
    lax.fori_loop(0, EK // 16, fill_ones, 0)
    _zero_vec(zbuf, ROWS_T + 16)
    _tile_node_init(zbuf, degsp, sid)
    plsc.subcore_barrier()

    def deg_body(j, _):
        g = j * NS + sid
        @pl.when(g < ECHUNKS)
        def _():
            base = cid * E + g * EK
            pltpu.sync_copy(dst2.at[pl.ds(base, EK)], dstbuf.at[0])
            pltpu.sync_copy(onesbuf, degsp.at[dstbuf.at[0]], add=True)
        return 0
    lax.fori_loop(0, EJ, deg_body, 0)

    def gat_body(j, _):
        g = j * NS + sid
        @pl.when(g < ICHUNKS)
        def _():
            base = cid * N + g * IK
            pltpu.sync_copy(ids2.at[pl.ds(base, IK)], idbuf)
            pltpu.async_copy(emb.at[idbuf], rowbuf, sem).wait()
            pltpu.sync_copy(rowbuf, h_out.at[pl.ds(base, IK)])
        return 0
    lax.fori_loop(0, IJ, gat_body, 0)

    plsc.subcore_barrier()
    _tile_node_out(degsp, deg_out, sid, cid * N, zbuf)


@functools.partial(
    pl.kernel,
    out_type=(jax.ShapeDtypeStruct((2 * N, D), jnp.float32),
              jax.ShapeDtypeStruct((2 * N,), jnp.float32)),
    mesh=_mesh,
    scratch_types=[
        pltpu.VMEM((1, EK), jnp.int32),
        pltpu.VMEM((EK,), jnp.int32),
        pltpu.VMEM((1, EK), jnp.int32),
        pltpu.VMEM((EK,), jnp.int32),
        pltpu.VMEM((EK, D), jnp.float32),
        pltpu.VMEM((EK,), jnp.float32),
        pltpu.VMEM((16, D), jnp.float32),
        pltpu.VMEM((ROWS_T + 16,), jnp.float32),
        pltpu.VMEM_SHARED((N, D), jnp.float32),
        pltpu.VMEM_SHARED((SPN,), jnp.float32),
        pltpu.SemaphoreType.DMA,
    ],
)
def _sc_segsum(src2, dst2, hwr2, r2, agg_out, t_out,
               srcbuf, srcgbuf, dstbuf, dstgbuf, rowbuf, valbuf,
               zrow, zbuf, aggsp, tsp, sem):
    cid = lax.axis_index("c")
    sid = lax.axis_index("s")

    def zrow_body(i, _):
        zrow[i // 8, pl.ds((i % 8) * 16, 16)] = jnp.zeros((16,), jnp.float32)
        return 0
    lax.fori_loop(0, 16 * (D // 16), zrow_body, 0)
    _zero_vec(zbuf, ROWS_T + 16)

    def zagg_body(i, _):
        pltpu.sync_copy(zrow, aggsp.at[pl.ds(sid * ROWS_T + i * 16, 16)])
        return 0
    lax.fori_loop(0, ROWS_T // 16, zagg_body, 0)
    @pl.when(sid == NS - 1)
    def _():
        pltpu.sync_copy(zrow, aggsp.at[pl.ds(NS * ROWS_T, 16)])
    _tile_node_init(zbuf, tsp, sid)
    plsc.subcore_barrier()

    coff = cid * N

    def edge_body(j, _):
        g = j * NS + sid
        @pl.when(g < ECHUNKS)
        def _():
            base = cid * E + g * EK
            pltpu.sync_copy(src2.at[pl.ds(base, EK)], srcbuf.at[0])
            pltpu.sync_copy(dst2.at[pl.ds(base, EK)], dstbuf.at[0])

            def idx_body(i, _):
                s16 = srcbuf[0, pl.ds(i * 16, 16)]
                srcgbuf[pl.ds(i * 16, 16)] = s16 + coff
                d16 = dstbuf[0, pl.ds(i * 16, 16)]
                dstgbuf[pl.ds(i * 16, 16)] = d16 + coff
                return 0
            lax.fori_loop(0, EK // 16, idx_body, 0)

            pltpu.async_copy(hwr2.at[srcgbuf], rowbuf, sem).wait()
            pltpu.async_copy(r2.at[dstgbuf], valbuf, sem).wait()
            pltpu.sync_copy(rowbuf, aggsp.at[dstbuf.at[0]], add=True)
            pltpu.sync_copy(valbuf, tsp.at[srcbuf.at[0]], add=True)
        return 0
    lax.fori_loop(0, EJ, edge_body, 0)

    plsc.subcore_barrier()
    def aout_body(i, _):
        row = sid * ROWS_T + i * 16
        pltpu.sync_copy(aggsp.at[pl.ds(row, 16)], rowbuf.at[pl.ds(0, 16)])
        pltpu.sync_copy(rowbuf.at[pl.ds(0, 16)],
                        agg_out.at[pl.ds(cid * N + row, 16)])
        return 0
    lax.fori_loop(0, ROWS_T // 16, aout_body, 0)
    @pl.when(sid == NS - 1)
    def _():
        pltpu.sync_copy(aggsp.at[pl.ds(NS * ROWS_T, 16)], rowbuf.at[pl.ds(0, 16)])
        pltpu.sync_copy(rowbuf.at[pl.ds(0, 16)],
                        agg_out.at[pl.ds(cid * N + NS * ROWS_T, 16)])
    _tile_node_out(tsp, t_out, sid, cid * N, zbuf)


def _tca_body(h_ref, deg_ref, w_ref, r_ref, hwr_ref):
    d = jnp.maximum(deg_ref[...], 1.0)
    r = lax.rsqrt(d)
    r_ref[...] = r
    for g in range(2):
        hw = jnp.dot(h_ref[g], w_ref[g], preferred_element_type=jnp.float32,
                     precision=lax.Precision.HIGHEST)
        hwr_ref[g] = r[g][:, None] * hw


_tca = pl.pallas_call(
    _tca_body,
    out_shape=(jax.ShapeDtypeStruct((2, N), jnp.float32),
               jax.ShapeDtypeStruct((2, N, D), jnp.float32)),
    compiler_params=pltpu.CompilerParams(vmem_limit_bytes=100 * 1024 * 1024),
)


def _tcb_body(agg_ref, r_ref, t_ref, w_ref, out_ref):
    for g in range(2):
        r = r_ref[g]
        x = r[:, None] * agg_ref[g]
        h1 = jnp.where(x >= 0, x, 0.01 * x)
        c = r * t_ref[g]
        s = jnp.dot(c[None, :], h1, preferred_element_type=jnp.float32,
                    precision=lax.Precision.HIGHEST)
        out_ref[g] = (jnp.dot(s, w_ref[g], preferred_element_type=jnp.float32,
                              precision=lax.Precision.HIGHEST) / float(N))[0]


_tcb = pl.pallas_call(
    _tcb_body,
    out_shape=jax.ShapeDtypeStruct((2, D), jnp.float32),
    compiler_params=pltpu.CompilerParams(vmem_limit_bytes=100 * 1024 * 1024),
)


def kernel(gu_ids, gu_edge_index, gv_ids, gv_edge_index, emb, Wp1, Wp2, Wc1, Wc2):
    ids2 = jnp.concatenate([gu_ids, gv_ids]).astype(jnp.int32)
    src2 = jnp.concatenate([gu_edge_index[0], gv_edge_index[0]]).astype(jnp.int32)
    dst2 = jnp.concatenate([gu_edge_index[1], gv_edge_index[1]]).astype(jnp.int32)
    emb = emb.astype(jnp.float32)

    h2, deg2 = _sc_deg_gather(ids2, dst2, emb)
    r, hwr = _tca(h2.reshape(2, N, D), deg2.reshape(2, N),
                  jnp.stack([Wp1, Wc1]))
    r_pad = jnp.pad(r.reshape(2 * N), (0, RPAD - 2 * N))
    agg2, t2 = _sc_segsum(src2, dst2, hwr.reshape(2 * N, D), r_pad)
    out = _tcb(agg2.reshape(2, N, D), r, t2.reshape(2, N),
               jnp.stack([Wp2, Wc2]))
    return out


# speedup vs baseline: 27.9383x; 1.6444x over previous
import functools

import jax
import jax.numpy as jnp
from jax import lax
from jax.experimental import pallas as pl
from jax.experimental.pallas import tpu as pltpu
from jax.experimental.pallas import tpu_sc as plsc

N = 10000
E = 320000
V = 50000
D = 128

NC = 2
NS = 16

ROWS_T = 624
EK = 128
ECHUNKS = E // EK
EJ = -(-ECHUNKS // NS)
IK = 80
ICHUNKS = N // IK
IJ = -(-ICHUNKS // NS)
NBUF = 2
SPN = 10112
RPAD = 20096

_mesh = plsc.VectorSubcoreMesh(core_axis_name="c", subcore_axis_name="s")


def _zero_vec(ref, nwords):
    def body(i, _):
        ref[pl.ds(i * 16, 16)] = jnp.zeros((16,), jnp.float32)
        return 0
    lax.fori_loop(0, nwords // 16, body, 0)


def _tile_node_init(zbuf, shared, sid):
    pltpu.sync_copy(zbuf.at[pl.ds(0, ROWS_T)], shared.at[pl.ds(sid * ROWS_T, ROWS_T)])
    @pl.when(sid == NS - 1)
    def _():
        pltpu.sync_copy(zbuf.at[pl.ds(0, 16)], shared.at[pl.ds(NS * ROWS_T, 16)])


def _tile_node_out(shared, out, sid, base, stage):
    pltpu.sync_copy(shared.at[pl.ds(sid * ROWS_T, ROWS_T)],
                    stage.at[pl.ds(0, ROWS_T)])
    pltpu.sync_copy(stage.at[pl.ds(0, ROWS_T)],
                    out.at[pl.ds(base + sid * ROWS_T, ROWS_T)])
    @pl.when(sid == NS - 1)
    def _():
        pltpu.sync_copy(shared.at[pl.ds(NS * ROWS_T, 16)], stage.at[pl.ds(0, 16)])
        pltpu.sync_copy(stage.at[pl.ds(0, 16)],
                        out.at[pl.ds(base + NS * ROWS_T, 16)])


@functools.partial(
    pl.kernel,
    out_type=(jax.ShapeDtypeStruct((2 * N, D), jnp.float32),
              jax.ShapeDtypeStruct((2 * N,), jnp.float32)),
    mesh=_mesh,
    scratch_types=[
        pltpu.VMEM((IK,), jnp.int32),
        pltpu.VMEM((IK, D), jnp.float32),
        pltpu.VMEM((1, EK), jnp.int32),
        pltpu.VMEM((EK,), jnp.float32),
        pltpu.VMEM((ROWS_T + 16,), jnp.float32),
        pltpu.VMEM_SHARED((SPN,), jnp.float32),
        pltpu.SemaphoreType.DMA,
    ],
)
def _sc_deg_gather(ids2, dst2, emb, h_out, deg_out,
                   idbuf, rowbuf, dstbuf, onesbuf, zbuf, degsp, sem):
    cid = lax.axis_index("c")
    sid = lax.axis_index("s")

    def fill_ones(i, _):
        onesbuf[pl.ds(i * 16, 16)] = jnp.full((16,), 1.0, jnp.float32)
        return 0
    lax.fori_loop(0, EK // 16, fill_ones, 0)
    _zero_vec(zbuf, ROWS_T + 16)
    _tile_node_init(zbuf, degsp, sid)
    plsc.subcore_barrier()

    def deg_body(j, _):
        g = j * NS + sid
        @pl.when(g < ECHUNKS)
        def _():
            base = cid * E + g * EK
            pltpu.sync_copy(dst2.at[pl.ds(base, EK)], dstbuf.at[0])
            pltpu.sync_copy(onesbuf, degsp.at[dstbuf.at[0]], add=True)
        return 0
    lax.fori_loop(0, EJ, deg_body, 0)

    def gat_body(j, _):
        g = j * NS + sid
        @pl.when(g < ICHUNKS)
        def _():
            base = cid * N + g * IK
            pltpu.sync_copy(ids2.at[pl.ds(base, IK)], idbuf)
            pltpu.async_copy(emb.at[idbuf], rowbuf, sem).wait()
            pltpu.sync_copy(rowbuf, h_out.at[pl.ds(base, IK)])
        return 0
    lax.fori_loop(0, IJ, gat_body, 0)

    plsc.subcore_barrier()
    _tile_node_out(degsp, deg_out, sid, cid * N, zbuf)


@functools.partial(
    pl.kernel,
    out_type=(jax.ShapeDtypeStruct((2 * N, D), jnp.float32),
              jax.ShapeDtypeStruct((2 * N,), jnp.float32)),
    mesh=_mesh,
    scratch_types=[
        pltpu.VMEM((NBUF, EK), jnp.int32),
        pltpu.VMEM((NBUF, EK), jnp.int32),
        pltpu.VMEM((NBUF, EK), jnp.int32),
        pltpu.VMEM((NBUF, EK), jnp.int32),
        pltpu.VMEM((NBUF, EK, D), jnp.float32),
        pltpu.VMEM((NBUF, EK), jnp.float32),
        pltpu.VMEM((16, D), jnp.float32),
        pltpu.VMEM((ROWS_T + 16,), jnp.float32),
        pltpu.VMEM_SHARED((N, D), jnp.float32),
        pltpu.VMEM_SHARED((SPN,), jnp.float32),
        pltpu.SemaphoreType.DMA,
        pltpu.SemaphoreType.DMA,
    ],
)
def _sc_segsum(src2, dst2, hwr2, r2, agg_out, t_out,
               srcbufs, srcgbufs, dstbufs, dstgbufs, rowbufs, valbufs,
               zrow, zbuf, aggsp, tsp, gsem, vsem):
    cid = lax.axis_index("c")
    sid = lax.axis_index("s")

    def zrow_body(i, _):
        zrow[i // 8, pl.ds((i % 8) * 16, 16)] = jnp.zeros((16,), jnp.float32)
        return 0
    lax.fori_loop(0, 16 * (D // 16), zrow_body, 0)
    _zero_vec(zbuf, ROWS_T + 16)

    def zagg_body(i, _):
        pltpu.sync_copy(zrow, aggsp.at[pl.ds(sid * ROWS_T + i * 16, 16)])
        return 0
    lax.fori_loop(0, ROWS_T // 16, zagg_body, 0)
    @pl.when(sid == NS - 1)
    def _():
        pltpu.sync_copy(zrow, aggsp.at[pl.ds(NS * ROWS_T, 16)])
    _tile_node_init(zbuf, tsp, sid)
    plsc.subcore_barrier()

    coff = cid * N

    def fire(j, b):
        g = j * NS + sid
        @pl.when(g < ECHUNKS)
        def _():
            base = cid * E + g * EK
            pltpu.sync_copy(src2.at[pl.ds(base, EK)], srcbufs.at[b])
            pltpu.sync_copy(dst2.at[pl.ds(base, EK)], dstbufs.at[b])

            def idx_body(i, _):
                s16 = srcbufs[b, pl.ds(i * 16, 16)]
                srcgbufs[b, pl.ds(i * 16, 16)] = s16 + coff
                d16 = dstbufs[b, pl.ds(i * 16, 16)]
                dstgbufs[b, pl.ds(i * 16, 16)] = d16 + coff
                return 0
            lax.fori_loop(0, EK // 16, idx_body, 0)
            pltpu.async_copy(hwr2.at[srcgbufs.at[b]], rowbufs.at[b], gsem)
            pltpu.async_copy(r2.at[dstgbufs.at[b]], valbufs.at[b], vsem)

    def consume(j, b):
        g = j * NS + sid
        @pl.when(g < ECHUNKS)
        def _():
            pltpu.make_async_copy(hwr2.at[pl.ds(0, EK)], rowbufs.at[b], gsem).wait()
            pltpu.make_async_copy(r2.at[pl.ds(0, EK)], valbufs.at[b], vsem).wait()
            pltpu.sync_copy(rowbufs.at[b], aggsp.at[dstbufs.at[b]], add=True)
            pltpu.sync_copy(valbufs.at[b], tsp.at[srcbufs.at[b]], add=True)

    for b in range(NBUF):
        fire(b, b)

    NGRP = -(-EJ // NBUF)

    def grp_body(grp, _):
        for b in range(NBUF):
            j = grp * NBUF + b
            consume(j, b)
            fire(j + NBUF, b)
        return 0
    lax.fori_loop(0, NGRP, grp_body, 0)

    plsc.subcore_barrier()
    def aout_body(i, _):
        row = sid * ROWS_T + i * 16
        pltpu.sync_copy(aggsp.at[pl.ds(row, 16)], rowbufs.at[0, pl.ds(0, 16)])
        pltpu.sync_copy(rowbufs.at[0, pl.ds(0, 16)],
                        agg_out.at[pl.ds(cid * N + row, 16)])
        return 0
    lax.fori_loop(0, ROWS_T // 16, aout_body, 0)
    @pl.when(sid == NS - 1)
    def _():
        pltpu.sync_copy(aggsp.at[pl.ds(NS * ROWS_T, 16)], rowbufs.at[0, pl.ds(0, 16)])
        pltpu.sync_copy(rowbufs.at[0, pl.ds(0, 16)],
                        agg_out.at[pl.ds(cid * N + NS * ROWS_T, 16)])
    _tile_node_out(tsp, t_out, sid, cid * N, zbuf)


def _tca_body(h_ref, deg_ref, w_ref, r_ref, hwr_ref):
    d = jnp.maximum(deg_ref[...], 1.0)
    r = lax.rsqrt(d)
    r_ref[...] = r
    for g in range(2):
        hw = jnp.dot(h_ref[g], w_ref[g], preferred_element_type=jnp.float32,
                     precision=lax.Precision.HIGHEST)
        hwr_ref[g] = r[g][:, None] * hw


_tca = pl.pallas_call(
    _tca_body,
    out_shape=(jax.ShapeDtypeStruct((2, N), jnp.float32),
               jax.ShapeDtypeStruct((2, N, D), jnp.float32)),
    compiler_params=pltpu.CompilerParams(vmem_limit_bytes=100 * 1024 * 1024),
)


def _tcb_body(agg_ref, r_ref, t_ref, w_ref, out_ref):
    for g in range(2):
        r = r_ref[g]
        x = r[:, None] * agg_ref[g]
        h1 = jnp.where(x >= 0, x, 0.01 * x)
        c = r * t_ref[g]
        s = jnp.dot(c[None, :], h1, preferred_element_type=jnp.float32,
                    precision=lax.Precision.HIGHEST)
        out_ref[g] = (jnp.dot(s, w_ref[g], preferred_element_type=jnp.float32,
                              precision=lax.Precision.HIGHEST) / float(N))[0]


_tcb = pl.pallas_call(
    _tcb_body,
    out_shape=jax.ShapeDtypeStruct((2, D), jnp.float32),
    compiler_params=pltpu.CompilerParams(vmem_limit_bytes=100 * 1024 * 1024),
)


def kernel(gu_ids, gu_edge_index, gv_ids, gv_edge_index, emb, Wp1, Wp2, Wc1, Wc2):
    ids2 = jnp.concatenate([gu_ids, gv_ids]).astype(jnp.int32)
    src2 = jnp.concatenate([gu_edge_index[0], gv_edge_index[0]]).astype(jnp.int32)
    dst2 = jnp.concatenate([gu_edge_index[1], gv_edge_index[1]]).astype(jnp.int32)
    emb = emb.astype(jnp.float32)

    h2, deg2 = _sc_deg_gather(ids2, dst2, emb)
    r, hwr = _tca(h2.reshape(2, N, D), deg2.reshape(2, N),
                  jnp.stack([Wp1, Wc1]))
    r_pad = jnp.pad(r.reshape(2 * N), (0, RPAD - 2 * N))
    agg2, t2 = _sc_segsum(src2, dst2, hwr.reshape(2 * N, D), r_pad)
    out = _tcb(agg2.reshape(2, N, D), r, t2.reshape(2, N),
               jnp.stack([Wp2, Wc2]))
    return out


# speedup vs baseline: 28.3005x; 1.0130x over previous
import functools

import jax
import jax.numpy as jnp
from jax import lax
from jax.experimental import pallas as pl
from jax.experimental.pallas import tpu as pltpu
from jax.experimental.pallas import tpu_sc as plsc

N = 10000
E = 320000
V = 50000
D = 128

NC = 2
NS = 16

ROWS_T = 624
EK = 128
ECHUNKS = E // EK
EJ = -(-ECHUNKS // NS)
IK = 80
ICHUNKS = N // IK
IJ = -(-ICHUNKS // NS)
NBUF = 2
SPN = 10112
RPAD = 20096

_mesh = plsc.VectorSubcoreMesh(core_axis_name="c", subcore_axis_name="s")


def _zero_vec(ref, nwords):
    def body(i, _):
        ref[pl.ds(i * 16, 16)] = jnp.zeros((16,), jnp.float32)
        return 0
    lax.fori_loop(0, nwords // 16, body, 0)


def _tile_node_init(zbuf, shared, sid):
    pltpu.sync_copy(zbuf.at[pl.ds(0, ROWS_T)], shared.at[pl.ds(sid * ROWS_T, ROWS_T)])
    @pl.when(sid == NS - 1)
    def _():
        pltpu.sync_copy(zbuf.at[pl.ds(0, 16)], shared.at[pl.ds(NS * ROWS_T, 16)])


def _tile_node_out(shared, out, sid, base, stage):
    pltpu.sync_copy(shared.at[pl.ds(sid * ROWS_T, ROWS_T)],
                    stage.at[pl.ds(0, ROWS_T)])
    pltpu.sync_copy(stage.at[pl.ds(0, ROWS_T)],
                    out.at[pl.ds(base + sid * ROWS_T, ROWS_T)])
    @pl.when(sid == NS - 1)
    def _():
        pltpu.sync_copy(shared.at[pl.ds(NS * ROWS_T, 16)], stage.at[pl.ds(0, 16)])
        pltpu.sync_copy(stage.at[pl.ds(0, 16)],
                        out.at[pl.ds(base + NS * ROWS_T, 16)])


@functools.partial(
    pl.kernel,
    out_type=(jax.ShapeDtypeStruct((2 * N, D), jnp.float32),
              jax.ShapeDtypeStruct((2 * N,), jnp.float32)),
    mesh=_mesh,
    scratch_types=[
        pltpu.VMEM((IK,), jnp.int32),
        pltpu.VMEM((IK, D), jnp.float32),
        pltpu.VMEM((1, EK), jnp.int32),
        pltpu.VMEM((EK,), jnp.float32),
        pltpu.VMEM((ROWS_T + 16,), jnp.float32),
        pltpu.VMEM_SHARED((SPN,), jnp.float32),
        pltpu.SemaphoreType.DMA,
    ],
)
def _sc_deg_gather(ids2, dst2, emb, h_out, deg_out,
                   idbuf, rowbuf, dstbuf, onesbuf, zbuf, degsp, sem):
    cid = lax.axis_index("c")
    sid = lax.axis_index("s")

    def fill_ones(i, _):
        onesbuf[pl.ds(i * 16, 16)] = jnp.full((16,), 1.0, jnp.float32)
        return 0
    lax.fori_loop(0, EK // 16, fill_ones, 0)
    _zero_vec(zbuf, ROWS_T + 16)
    _tile_node_init(zbuf, degsp, sid)
    plsc.subcore_barrier()

    def deg_body(j, _):
        g = j * NS + sid
        @pl.when(g < ECHUNKS)
        def _():
            base = cid * E + g * EK
            pltpu.sync_copy(dst2.at[pl.ds(base, EK)], dstbuf.at[0])
            pltpu.sync_copy(onesbuf, degsp.at[dstbuf.at[0]], add=True)
        return 0
    lax.fori_loop(0, EJ, deg_body, 0)

    def gat_body(j, _):
        g = j * NS + sid
        @pl.when(g < ICHUNKS)
        def _():
            base = cid * N + g * IK
            pltpu.sync_copy(ids2.at[pl.ds(base, IK)], idbuf)
            pltpu.async_copy(emb.at[idbuf], rowbuf, sem).wait()
            pltpu.sync_copy(rowbuf, h_out.at[pl.ds(base, IK)])
        return 0
    lax.fori_loop(0, IJ, gat_body, 0)

    plsc.subcore_barrier()
    _tile_node_out(degsp, deg_out, sid, cid * N, zbuf)


@functools.partial(
    pl.kernel,
    out_type=(jax.ShapeDtypeStruct((2 * N, D), jnp.float32),
              jax.ShapeDtypeStruct((2 * N,), jnp.float32)),
    mesh=_mesh,
    scratch_types=[
        pltpu.VMEM((NBUF, EK), jnp.int32),
        pltpu.VMEM((NBUF, EK), jnp.int32),
        pltpu.VMEM((NBUF, EK), jnp.int32),
        pltpu.VMEM((NBUF, EK), jnp.int32),
        pltpu.VMEM((NBUF, EK, D), jnp.float32),
        pltpu.VMEM((NBUF, EK), jnp.float32),
        pltpu.VMEM((16, D), jnp.float32),
        pltpu.VMEM((ROWS_T + 16,), jnp.float32),
        pltpu.VMEM_SHARED((N, D), jnp.float32),
        pltpu.VMEM_SHARED((SPN,), jnp.float32),
        pltpu.SemaphoreType.DMA,
        pltpu.SemaphoreType.DMA,
        pltpu.SemaphoreType.DMA,
        pltpu.SemaphoreType.DMA,
    ],
)
def _sc_segsum(src2, dst2, hwr2, r2, agg_out, t_out,
               srcbufs, srcgbufs, dstbufs, dstgbufs, rowbufs, valbufs,
               zrow, zbuf, aggsp, tsp, gsem, vsem, ssem, tsem):
    cid = lax.axis_index("c")
    sid = lax.axis_index("s")

    def zrow_body(i, _):
        zrow[i // 8, pl.ds((i % 8) * 16, 16)] = jnp.zeros((16,), jnp.float32)
        return 0
    lax.fori_loop(0, 16 * (D // 16), zrow_body, 0)
    _zero_vec(zbuf, ROWS_T + 16)

    def zagg_body(i, _):
        pltpu.sync_copy(zrow, aggsp.at[pl.ds(sid * ROWS_T + i * 16, 16)])
        return 0
    lax.fori_loop(0, ROWS_T // 16, zagg_body, 0)
    @pl.when(sid == NS - 1)
    def _():
        pltpu.sync_copy(zrow, aggsp.at[pl.ds(NS * ROWS_T, 16)])
    _tile_node_init(zbuf, tsp, sid)
    plsc.subcore_barrier()

    coff = cid * N

    def drain_scatter(b):
        pltpu.make_async_copy(rowbufs.at[b], aggsp.at[dstbufs.at[b]], ssem).wait()
        pltpu.make_async_copy(valbufs.at[b], tsp.at[srcbufs.at[b]], tsem).wait()

    def fire(j, b, drain):
        g = j * NS + sid
        if drain:
            @pl.when((j - NBUF) * NS + sid < ECHUNKS)
            def _():
                drain_scatter(b)
        @pl.when(g < ECHUNKS)
        def _():
            base = cid * E + g * EK
            pltpu.sync_copy(src2.at[pl.ds(base, EK)], srcbufs.at[b])
            pltpu.sync_copy(dst2.at[pl.ds(base, EK)], dstbufs.at[b])

            def idx_body(i, _):
                s16 = srcbufs[b, pl.ds(i * 16, 16)]
                srcgbufs[b, pl.ds(i * 16, 16)] = s16 + coff
                d16 = dstbufs[b, pl.ds(i * 16, 16)]
                dstgbufs[b, pl.ds(i * 16, 16)] = d16 + coff
                return 0
            lax.fori_loop(0, EK // 16, idx_body, 0)
            pltpu.async_copy(hwr2.at[srcgbufs.at[b]], rowbufs.at[b], gsem)
            pltpu.async_copy(r2.at[dstgbufs.at[b]], valbufs.at[b], vsem)

    def consume(j, b):
        g = j * NS + sid
        @pl.when(g < ECHUNKS)
        def _():
            pltpu.make_async_copy(hwr2.at[pl.ds(0, EK)], rowbufs.at[b], gsem).wait()
            pltpu.make_async_copy(r2.at[pl.ds(0, EK)], valbufs.at[b], vsem).wait()
            pltpu.async_copy(rowbufs.at[b], aggsp.at[dstbufs.at[b]], ssem, add=True)
            pltpu.async_copy(valbufs.at[b], tsp.at[srcbufs.at[b]], tsem, add=True)

    for b in range(NBUF):
        fire(b, b, drain=False)

    NGRP = -(-EJ // NBUF)

    def grp_body(grp, _):
        for b in range(NBUF):
            j = grp * NBUF + b
            consume(j, b)
            fire(j + NBUF, b, drain=True)
        return 0
    lax.fori_loop(0, NGRP, grp_body, 0)

    plsc.subcore_barrier()
    def aout_body(i, _):
        row = sid * ROWS_T + i * 16
        pltpu.sync_copy(aggsp.at[pl.ds(row, 16)], rowbufs.at[0, pl.ds(0, 16)])
        pltpu.sync_copy(rowbufs.at[0, pl.ds(0, 16)],
                        agg_out.at[pl.ds(cid * N + row, 16)])
        return 0
    lax.fori_loop(0, ROWS_T // 16, aout_body, 0)
    @pl.when(sid == NS - 1)
    def _():
        pltpu.sync_copy(aggsp.at[pl.ds(NS * ROWS_T, 16)], rowbufs.at[0, pl.ds(0, 16)])
        pltpu.sync_copy(rowbufs.at[0, pl.ds(0, 16)],
                        agg_out.at[pl.ds(cid * N + NS * ROWS_T, 16)])
    _tile_node_out(tsp, t_out, sid, cid * N, zbuf)


def _tca_body(h_ref, deg_ref, w_ref, r_ref, hwr_ref):
    d = jnp.maximum(deg_ref[...], 1.0)
    r = lax.rsqrt(d)
    r_ref[...] = r
    for g in range(2):
        hw = jnp.dot(h_ref[g], w_ref[g], preferred_element_type=jnp.float32,
                     precision=lax.Precision.HIGHEST)
        hwr_ref[g] = r[g][:, None] * hw


_tca = pl.pallas_call(
    _tca_body,
    out_shape=(jax.ShapeDtypeStruct((2, N), jnp.float32),
               jax.ShapeDtypeStruct((2, N, D), jnp.float32)),
    compiler_params=pltpu.CompilerParams(vmem_limit_bytes=100 * 1024 * 1024),
)


def _tcb_body(agg_ref, r_ref, t_ref, w_ref, out_ref):
    for g in range(2):
        r = r_ref[g]
        x = r[:, None] * agg_ref[g]
        h1 = jnp.where(x >= 0, x, 0.01 * x)
        c = r * t_ref[g]
        s = jnp.dot(c[None, :], h1, preferred_element_type=jnp.float32,
                    precision=lax.Precision.HIGHEST)
        out_ref[g] = (jnp.dot(s, w_ref[g], preferred_element_type=jnp.float32,
                              precision=lax.Precision.HIGHEST) / float(N))[0]


_tcb = pl.pallas_call(
    _tcb_body,
    out_shape=jax.ShapeDtypeStruct((2, D), jnp.float32),
    compiler_params=pltpu.CompilerParams(vmem_limit_bytes=100 * 1024 * 1024),
)


def kernel(gu_ids, gu_edge_index, gv_ids, gv_edge_index, emb, Wp1, Wp2, Wc1, Wc2):
    ids2 = jnp.concatenate([gu_ids, gv_ids]).astype(jnp.int32)
    src2 = jnp.concatenate([gu_edge_index[0], gv_edge_index[0]]).astype(jnp.int32)
    dst2 = jnp.concatenate([gu_edge_index[1], gv_edge_index[1]]).astype(jnp.int32)
    emb = emb.astype(jnp.float32)

    h2, deg2 = _sc_deg_gather(ids2, dst2, emb)
    r, hwr = _tca(h2.reshape(2, N, D), deg2.reshape(2, N),
                  jnp.stack([Wp1, Wc1]))
    r_pad = jnp.pad(r.reshape(2 * N), (0, RPAD - 2 * N))
    agg2, t2 = _sc_segsum(src2, dst2, hwr.reshape(2 * N, D), r_pad)
    out = _tcb(agg2.reshape(2, N, D), r, t2.reshape(2, N),
               jnp.stack([Wp2, Wc2]))
    return out


# speedup vs baseline: 33.0125x; 1.1665x over previous
import functools

import jax
import jax.numpy as jnp
from jax import lax
from jax.experimental import pallas as pl
from jax.experimental.pallas import tpu as pltpu
from jax.experimental.pallas import tpu_sc as plsc

N = 10000
E = 320000
V = 50000
D = 128

NC = 2
NS = 16

ROWS_T = 624
EK = 128
ECHUNKS = E // EK
EJ = -(-ECHUNKS // NS)
IK = 80
ICHUNKS = N // IK
IJ = -(-ICHUNKS // NS)
NBUF = 2
IB = 6
SPN = 10112
RPAD = 20096

_mesh = plsc.VectorSubcoreMesh(core_axis_name="c", subcore_axis_name="s")


def _zero_vec(ref, nwords):
    def body(i, _):
        ref[pl.ds(i * 16, 16)] = jnp.zeros((16,), jnp.float32)
        return 0
    lax.fori_loop(0, nwords // 16, body, 0)


def _tile_node_init(zbuf, shared, sid):
    pltpu.sync_copy(zbuf.at[pl.ds(0, ROWS_T)], shared.at[pl.ds(sid * ROWS_T, ROWS_T)])
    @pl.when(sid == NS - 1)
    def _():
        pltpu.sync_copy(zbuf.at[pl.ds(0, 16)], shared.at[pl.ds(NS * ROWS_T, 16)])


def _tile_node_out(shared, out, sid, base, stage):
    pltpu.sync_copy(shared.at[pl.ds(sid * ROWS_T, ROWS_T)],
                    stage.at[pl.ds(0, ROWS_T)])
    pltpu.sync_copy(stage.at[pl.ds(0, ROWS_T)],
                    out.at[pl.ds(base + sid * ROWS_T, ROWS_T)])
    @pl.when(sid == NS - 1)
    def _():
        pltpu.sync_copy(shared.at[pl.ds(NS * ROWS_T, 16)], stage.at[pl.ds(0, 16)])
        pltpu.sync_copy(stage.at[pl.ds(0, 16)],
                        out.at[pl.ds(base + NS * ROWS_T, 16)])


@functools.partial(
    pl.kernel,
    out_type=(jax.ShapeDtypeStruct((2 * N, D), jnp.float32),
              jax.ShapeDtypeStruct((2 * N,), jnp.float32)),
    mesh=_mesh,
    scratch_types=[
        pltpu.VMEM((IK,), jnp.int32),
        pltpu.VMEM((IK, D), jnp.float32),
        pltpu.VMEM((1, EK), jnp.int32),
        pltpu.VMEM((EK,), jnp.float32),
        pltpu.VMEM((ROWS_T + 16,), jnp.float32),
        pltpu.VMEM_SHARED((SPN,), jnp.float32),
        pltpu.SemaphoreType.DMA,
    ],
)
def _sc_deg_gather(ids2, dst2, emb, h_out, deg_out,
                   idbuf, rowbuf, dstbuf, onesbuf, zbuf, degsp, sem):
    cid = lax.axis_index("c")
    sid = lax.axis_index("s")

    def fill_ones(i, _):
        onesbuf[pl.ds(i * 16, 16)] = jnp.full((16,), 1.0, jnp.float32)
        return 0
    lax.fori_loop(0, EK // 16, fill_ones, 0)
    _zero_vec(zbuf, ROWS_T + 16)
    _tile_node_init(zbuf, degsp, sid)
    plsc.subcore_barrier()

    def deg_body(j, _):
        g = j * NS + sid
        @pl.when(g < ECHUNKS)
        def _():
            base = cid * E + g * EK
            pltpu.sync_copy(dst2.at[pl.ds(base, EK)], dstbuf.at[0])
            pltpu.sync_copy(onesbuf, degsp.at[dstbuf.at[0]], add=True)
        return 0
    lax.fori_loop(0, EJ, deg_body, 0)

    def gat_body(j, _):
        g = j * NS + sid
        @pl.when(g < ICHUNKS)
        def _():
            base = cid * N + g * IK
            pltpu.sync_copy(ids2.at[pl.ds(base, IK)], idbuf)
            pltpu.async_copy(emb.at[idbuf], rowbuf, sem).wait()
            pltpu.sync_copy(rowbuf, h_out.at[pl.ds(base, IK)])
        return 0
    lax.fori_loop(0, IJ, gat_body, 0)

    plsc.subcore_barrier()
    _tile_node_out(degsp, deg_out, sid, cid * N, zbuf)


@functools.partial(
    pl.kernel,
    out_type=(jax.ShapeDtypeStruct((2 * N, D), jnp.float32),
              jax.ShapeDtypeStruct((2 * N,), jnp.float32)),
    mesh=_mesh,
    scratch_types=[
        pltpu.VMEM((IB, EK), jnp.int32),
        pltpu.VMEM((IB, EK), jnp.int32),
        pltpu.VMEM((IB, EK), jnp.int32),
        pltpu.VMEM((IB, EK), jnp.int32),
        pltpu.VMEM((NBUF, EK, D), jnp.float32),
        pltpu.VMEM((NBUF, EK), jnp.float32),
        pltpu.VMEM((16, D), jnp.float32),
        pltpu.VMEM((ROWS_T + 16,), jnp.float32),
        pltpu.VMEM_SHARED((N, D), jnp.float32),
        pltpu.VMEM_SHARED((SPN,), jnp.float32),
        pltpu.SemaphoreType.DMA,
        pltpu.SemaphoreType.DMA,
        pltpu.SemaphoreType.DMA,
        pltpu.SemaphoreType.DMA,
        pltpu.SemaphoreType.DMA,
    ],
)
def _sc_segsum(src2, dst2, hwr2, r2, agg_out, t_out,
               srcbufs, srcgbufs, dstbufs, dstgbufs, rowbufs, valbufs,
               zrow, zbuf, aggsp, tsp, isem, gsem, vsem, ssem, tsem):
    cid = lax.axis_index("c")
    sid = lax.axis_index("s")

    def zrow_body(i, _):
        zrow[i // 8, pl.ds((i % 8) * 16, 16)] = jnp.zeros((16,), jnp.float32)
        return 0
    lax.fori_loop(0, 16 * (D // 16), zrow_body, 0)
    _zero_vec(zbuf, ROWS_T + 16)

    def zagg_body(i, _):
        pltpu.sync_copy(zrow, aggsp.at[pl.ds(sid * ROWS_T + i * 16, 16)])
        return 0
    lax.fori_loop(0, ROWS_T // 16, zagg_body, 0)
    @pl.when(sid == NS - 1)
    def _():
        pltpu.sync_copy(zrow, aggsp.at[pl.ds(NS * ROWS_T, 16)])
    _tile_node_init(zbuf, tsp, sid)
    plsc.subcore_barrier()

    coff = cid * N

    def fire_idx(k, ib):
        g = k * NS + sid
        @pl.when(g < ECHUNKS)
        def _():
            base = cid * E + g * EK
            pltpu.async_copy(src2.at[pl.ds(base, EK)], srcbufs.at[ib], isem)
            pltpu.async_copy(dst2.at[pl.ds(base, EK)], dstbufs.at[ib], isem)

    def fire_gather(k, ib, rb, drain):
        if drain:
            @pl.when((k - NBUF) * NS + sid < ECHUNKS)
            def _():
                pltpu.make_async_copy(rowbufs.at[rb], aggsp.at[dstbufs.at[ib]],
                                      ssem).wait()
                pltpu.make_async_copy(valbufs.at[rb], tsp.at[srcbufs.at[ib]],
                                      tsem).wait()
        g = k * NS + sid
        @pl.when(g < ECHUNKS)
        def _():
            pltpu.make_async_copy(src2.at[pl.ds(0, EK)], srcbufs.at[ib], isem).wait()
            pltpu.make_async_copy(src2.at[pl.ds(0, EK)], dstbufs.at[ib], isem).wait()

            def idx_body(i, _):
                s16 = srcbufs[ib, pl.ds(i * 16, 16)]
                srcgbufs[ib, pl.ds(i * 16, 16)] = s16 + coff
                d16 = dstbufs[ib, pl.ds(i * 16, 16)]
                dstgbufs[ib, pl.ds(i * 16, 16)] = d16 + coff
                return 0
            lax.fori_loop(0, EK // 16, idx_body, 0)
            pltpu.async_copy(hwr2.at[srcgbufs.at[ib]], rowbufs.at[rb], gsem)
            pltpu.async_copy(r2.at[dstgbufs.at[ib]], valbufs.at[rb], vsem)

    def consume(k, ib, rb):
        g = k * NS + sid
        @pl.when(g < ECHUNKS)
        def _():
            pltpu.make_async_copy(hwr2.at[pl.ds(0, EK)], rowbufs.at[rb], gsem).wait()
            pltpu.make_async_copy(r2.at[pl.ds(0, EK)], valbufs.at[rb], vsem).wait()
            pltpu.async_copy(rowbufs.at[rb], aggsp.at[dstbufs.at[ib]], ssem,
                             add=True)
            pltpu.async_copy(valbufs.at[rb], tsp.at[srcbufs.at[ib]], tsem,
                             add=True)

    for k in range(4):
        fire_idx(k, k % IB)
    for k in range(NBUF):
        fire_gather(k, k % IB, k % NBUF, drain=False)

    GRP = 6
    NGRP = -(-EJ // GRP)

    def grp_body(grp, _):
        j0 = grp * GRP
        for b6 in range(GRP):
            j = j0 + b6
            consume(j, b6 % IB, b6 % NBUF)
            fire_gather(j + NBUF, (b6 + NBUF) % IB, b6 % NBUF, drain=True)
            fire_idx(j + 4, (b6 + 4) % IB)
        return 0
    lax.fori_loop(0, NGRP, grp_body, 0)


    plsc.subcore_barrier()
    def aout_body(i, _):
        row = sid * ROWS_T + i * 16
        pltpu.sync_copy(aggsp.at[pl.ds(row, 16)], rowbufs.at[0, pl.ds(0, 16)])
        pltpu.sync_copy(rowbufs.at[0, pl.ds(0, 16)],
                        agg_out.at[pl.ds(cid * N + row, 16)])
        return 0
    lax.fori_loop(0, ROWS_T // 16, aout_body, 0)
    @pl.when(sid == NS - 1)
    def _():
        pltpu.sync_copy(aggsp.at[pl.ds(NS * ROWS_T, 16)], rowbufs.at[0, pl.ds(0, 16)])
        pltpu.sync_copy(rowbufs.at[0, pl.ds(0, 16)],
                        agg_out.at[pl.ds(cid * N + NS * ROWS_T, 16)])
    _tile_node_out(tsp, t_out, sid, cid * N, zbuf)


def _tca_body(h_ref, deg_ref, w_ref, r_ref, hwr_ref):
    d = jnp.maximum(deg_ref[...], 1.0)
    r = lax.rsqrt(d)
    r_ref[...] = r
    for g in range(2):
        hw = jnp.dot(h_ref[g], w_ref[g], preferred_element_type=jnp.float32,
                     precision=lax.Precision.HIGHEST)
        hwr_ref[g] = r[g][:, None] * hw


_tca = pl.pallas_call(
    _tca_body,
    out_shape=(jax.ShapeDtypeStruct((2, N), jnp.float32),
               jax.ShapeDtypeStruct((2, N, D), jnp.float32)),
    compiler_params=pltpu.CompilerParams(vmem_limit_bytes=100 * 1024 * 1024),
)


def _tcb_body(agg_ref, r_ref, t_ref, w_ref, out_ref):
    for g in range(2):
        r = r_ref[g]
        x = r[:, None] * agg_ref[g]
        h1 = jnp.where(x >= 0, x, 0.01 * x)
        c = r * t_ref[g]
        s = jnp.dot(c[None, :], h1, preferred_element_type=jnp.float32,
                    precision=lax.Precision.HIGHEST)
        out_ref[g] = (jnp.dot(s, w_ref[g], preferred_element_type=jnp.float32,
                              precision=lax.Precision.HIGHEST) / float(N))[0]


_tcb = pl.pallas_call(
    _tcb_body,
    out_shape=jax.ShapeDtypeStruct((2, D), jnp.float32),
    compiler_params=pltpu.CompilerParams(vmem_limit_bytes=100 * 1024 * 1024),
)


def kernel(gu_ids, gu_edge_index, gv_ids, gv_edge_index, emb, Wp1, Wp2, Wc1, Wc2):
    ids2 = jnp.concatenate([gu_ids, gv_ids]).astype(jnp.int32)
    src2 = jnp.concatenate([gu_edge_index[0], gv_edge_index[0]]).astype(jnp.int32)
    dst2 = jnp.concatenate([gu_edge_index[1], gv_edge_index[1]]).astype(jnp.int32)
    emb = emb.astype(jnp.float32)

    h2, deg2 = _sc_deg_gather(ids2, dst2, emb)
    r, hwr = _tca(h2.reshape(2, N, D), deg2.reshape(2, N),
                  jnp.stack([Wp1, Wc1]))
    r_pad = jnp.pad(r.reshape(2 * N), (0, RPAD - 2 * N))
    agg2, t2 = _sc_segsum(src2, dst2, hwr.reshape(2 * N, D), r_pad)
    out = _tcb(agg2.reshape(2, N, D), r, t2.reshape(2, N),
               jnp.stack([Wp2, Wc2]))
    return out


# speedup vs baseline: 40.4992x; 1.2268x over previous
import functools

import jax
import jax.numpy as jnp
from jax import lax
from jax.experimental import pallas as pl
from jax.experimental.pallas import tpu as pltpu
from jax.experimental.pallas import tpu_sc as plsc

N = 10000
E = 320000
V = 50000
D = 128

NC = 2
NS = 16

ROWS_T = 624
EK = 128
ECHUNKS = E // EK
EJ = -(-ECHUNKS // NS)
IK = 80
ICHUNKS = N // IK
IJ = -(-ICHUNKS // NS)
NBUF = 2
IB = 6
SPN = 10112
RPAD = 20096

_mesh = plsc.VectorSubcoreMesh(core_axis_name="c", subcore_axis_name="s")


def _zero_vec(ref, nwords):
    def body(i, _):
        ref[pl.ds(i * 16, 16)] = jnp.zeros((16,), jnp.float32)
        return 0
    lax.fori_loop(0, nwords // 16, body, 0)


def _tile_node_init(zbuf, shared, sid):
    pltpu.sync_copy(zbuf.at[pl.ds(0, ROWS_T)], shared.at[pl.ds(sid * ROWS_T, ROWS_T)])
    @pl.when(sid == NS - 1)
    def _():
        pltpu.sync_copy(zbuf.at[pl.ds(0, 16)], shared.at[pl.ds(NS * ROWS_T, 16)])


def _tile_node_out(shared, out, sid, base, stage):
    pltpu.sync_copy(shared.at[pl.ds(sid * ROWS_T, ROWS_T)],
                    stage.at[pl.ds(0, ROWS_T)])
    pltpu.sync_copy(stage.at[pl.ds(0, ROWS_T)],
                    out.at[pl.ds(base + sid * ROWS_T, ROWS_T)])
    @pl.when(sid == NS - 1)
    def _():
        pltpu.sync_copy(shared.at[pl.ds(NS * ROWS_T, 16)], stage.at[pl.ds(0, 16)])
        pltpu.sync_copy(stage.at[pl.ds(0, 16)],
                        out.at[pl.ds(base + NS * ROWS_T, 16)])


@functools.partial(
    pl.kernel,
    out_type=(jax.ShapeDtypeStruct((2 * N, D), jnp.float32),
              jax.ShapeDtypeStruct((2 * N,), jnp.float32)),
    mesh=_mesh,
    scratch_types=[
        pltpu.VMEM((2, IK), jnp.int32),
        pltpu.VMEM((2, IK, D), jnp.float32),
        pltpu.VMEM((8, EK), jnp.int32),
        pltpu.VMEM((EK,), jnp.float32),
        pltpu.VMEM((ROWS_T + 16,), jnp.float32),
        pltpu.VMEM_SHARED((SPN,), jnp.float32),
        pltpu.SemaphoreType.DMA,
        pltpu.SemaphoreType.DMA,
        pltpu.SemaphoreType.DMA,
    ],
)
def _sc_deg_gather(ids2, dst2, emb, h_out, deg_out,
                   idbufs, rowbufs, dstbufs, onesbuf, zbuf, degsp,
                   isem, asem, gsem):
    cid = lax.axis_index("c")
    sid = lax.axis_index("s")

    def fill_ones(i, _):
        onesbuf[pl.ds(i * 16, 16)] = jnp.full((16,), 1.0, jnp.float32)
        return 0
    lax.fori_loop(0, EK // 16, fill_ones, 0)
    _zero_vec(zbuf, ROWS_T + 16)
    _tile_node_init(zbuf, degsp, sid)
    plsc.subcore_barrier()

    def deg_grp(grp, _):
        for b in range(8):
            g = (grp * 8 + b) * NS + sid
            @pl.when(g < ECHUNKS)
            def _():
                pltpu.async_copy(dst2.at[pl.ds(cid * E + g * EK, EK)],
                                 dstbufs.at[b], isem)
        for b in range(8):
            g = (grp * 8 + b) * NS + sid
            @pl.when(g < ECHUNKS)
            def _():
                pltpu.make_async_copy(dst2.at[pl.ds(0, EK)], dstbufs.at[b],
                                      isem).wait()
        for b in range(8):
            g = (grp * 8 + b) * NS + sid
            @pl.when(g < ECHUNKS)
            def _():
                pltpu.async_copy(onesbuf, degsp.at[dstbufs.at[b]], asem,
                                 add=True)
        for b in range(8):
            g = (grp * 8 + b) * NS + sid
            @pl.when(g < ECHUNKS)
            def _():
                pltpu.make_async_copy(onesbuf, degsp.at[dstbufs.at[b]],
                                      asem).wait()
        return 0
    lax.fori_loop(0, -(-EJ // 8), deg_grp, 0)

    def g_fire(k, b):
        g = k * NS + sid
        @pl.when(g < ICHUNKS)
        def _():
            base = cid * N + g * IK
            pltpu.sync_copy(ids2.at[pl.ds(base, IK)], idbufs.at[b])
            pltpu.async_copy(emb.at[idbufs.at[b]], rowbufs.at[b], gsem)

    def g_consume(k, b):
        g = k * NS + sid
        @pl.when(g < ICHUNKS)
        def _():
            pltpu.make_async_copy(emb.at[pl.ds(0, IK)], rowbufs.at[b],
                                  gsem).wait()
            pltpu.sync_copy(rowbufs.at[b], h_out.at[pl.ds(cid * N + g * IK, IK)])

    g_fire(0, 0)
    g_fire(1, 1)
    for j in range(IJ):
        g_consume(j, j % 2)
        g_fire(j + 2, j % 2)

    plsc.subcore_barrier()
    _tile_node_out(degsp, deg_out, sid, cid * N, zbuf)


@functools.partial(
    pl.kernel,
    out_type=(jax.ShapeDtypeStruct((2 * N, D), jnp.float32),
              jax.ShapeDtypeStruct((2 * N,), jnp.float32)),
    mesh=_mesh,
    scratch_types=[
        pltpu.VMEM((IB, EK), jnp.int32),
        pltpu.VMEM((IB, EK), jnp.int32),
        pltpu.VMEM((IB, EK), jnp.int32),
        pltpu.VMEM((IB, EK), jnp.int32),
        pltpu.VMEM((NBUF, EK, D), jnp.float32),
        pltpu.VMEM((NBUF, EK), jnp.float32),
        pltpu.VMEM((16, D), jnp.float32),
        pltpu.VMEM((ROWS_T + 16,), jnp.float32),
        pltpu.VMEM_SHARED((N, D), jnp.float32),
        pltpu.VMEM_SHARED((SPN,), jnp.float32),
        pltpu.SemaphoreType.DMA,
        pltpu.SemaphoreType.DMA,
        pltpu.SemaphoreType.DMA,
        pltpu.SemaphoreType.DMA,
        pltpu.SemaphoreType.DMA,
    ],
)
def _sc_segsum(src2, dst2, hwr2, r2, agg_out, t_out,
               srcbufs, srcgbufs, dstbufs, dstgbufs, rowbufs, valbufs,
               zrow, zbuf, aggsp, tsp, isem, gsem, vsem, ssem, tsem):
    cid = lax.axis_index("c")
    sid = lax.axis_index("s")

    def zrow_body(i, _):
        zrow[i // 8, pl.ds((i % 8) * 16, 16)] = jnp.zeros((16,), jnp.float32)
        return 0
    lax.fori_loop(0, 16 * (D // 16), zrow_body, 0)
    _zero_vec(zbuf, ROWS_T + 16)

    def zagg_body(i, _):
        pltpu.sync_copy(zrow, aggsp.at[pl.ds(sid * ROWS_T + i * 16, 16)])
        return 0
    lax.fori_loop(0, ROWS_T // 16, zagg_body, 0)
    @pl.when(sid == NS - 1)
    def _():
        pltpu.sync_copy(zrow, aggsp.at[pl.ds(NS * ROWS_T, 16)])
    _tile_node_init(zbuf, tsp, sid)
    plsc.subcore_barrier()

    coff = cid * N

    def fire_idx(k, ib):
        g = k * NS + sid
        @pl.when(g < ECHUNKS)
        def _():
            base = cid * E + g * EK
            pltpu.async_copy(src2.at[pl.ds(base, EK)], srcbufs.at[ib], isem)
            pltpu.async_copy(dst2.at[pl.ds(base, EK)], dstbufs.at[ib], isem)

    def fire_gather(k, ib, rb, drain):
        if drain:
            @pl.when((k - NBUF) * NS + sid < ECHUNKS)
            def _():
                pltpu.make_async_copy(rowbufs.at[rb], aggsp.at[dstbufs.at[ib]],
                                      ssem).wait()
                pltpu.make_async_copy(valbufs.at[rb], tsp.at[srcbufs.at[ib]],
                                      tsem).wait()
        g = k * NS + sid
        @pl.when(g < ECHUNKS)
        def _():
            pltpu.make_async_copy(src2.at[pl.ds(0, EK)], srcbufs.at[ib], isem).wait()
            pltpu.make_async_copy(src2.at[pl.ds(0, EK)], dstbufs.at[ib], isem).wait()

            def idx_body(i, _):
                s16 = srcbufs[ib, pl.ds(i * 16, 16)]
                srcgbufs[ib, pl.ds(i * 16, 16)] = s16 + coff
                d16 = dstbufs[ib, pl.ds(i * 16, 16)]
                dstgbufs[ib, pl.ds(i * 16, 16)] = d16 + coff
                return 0
            lax.fori_loop(0, EK // 16, idx_body, 0)
            pltpu.async_copy(hwr2.at[srcgbufs.at[ib]], rowbufs.at[rb], gsem)
            pltpu.async_copy(r2.at[dstgbufs.at[ib]], valbufs.at[rb], vsem)

    def consume(k, ib, rb):
        g = k * NS + sid
        @pl.when(g < ECHUNKS)
        def _():
            pltpu.make_async_copy(hwr2.at[pl.ds(0, EK)], rowbufs.at[rb], gsem).wait()
            pltpu.make_async_copy(r2.at[pl.ds(0, EK)], valbufs.at[rb], vsem).wait()
            pltpu.async_copy(rowbufs.at[rb], aggsp.at[dstbufs.at[ib]], ssem,
                             add=True)
            pltpu.async_copy(valbufs.at[rb], tsp.at[srcbufs.at[ib]], tsem,
                             add=True)

    for k in range(4):
        fire_idx(k, k % IB)
    for k in range(NBUF):
        fire_gather(k, k % IB, k % NBUF, drain=False)

    GRP = 6
    NGRP = -(-EJ // GRP)

    def grp_body(grp, _):
        j0 = grp * GRP
        for b6 in range(GRP):
            j = j0 + b6
            consume(j, b6 % IB, b6 % NBUF)
            fire_gather(j + NBUF, (b6 + NBUF) % IB, b6 % NBUF, drain=True)
            fire_idx(j + 4, (b6 + 4) % IB)
        return 0
    lax.fori_loop(0, NGRP, grp_body, 0)


    plsc.subcore_barrier()
    def aout_body(i, _):
        row = sid * ROWS_T + i * 16
        pltpu.sync_copy(aggsp.at[pl.ds(row, 16)], rowbufs.at[0, pl.ds(0, 16)])
        pltpu.sync_copy(rowbufs.at[0, pl.ds(0, 16)],
                        agg_out.at[pl.ds(cid * N + row, 16)])
        return 0
    lax.fori_loop(0, ROWS_T // 16, aout_body, 0)
    @pl.when(sid == NS - 1)
    def _():
        pltpu.sync_copy(aggsp.at[pl.ds(NS * ROWS_T, 16)], rowbufs.at[0, pl.ds(0, 16)])
        pltpu.sync_copy(rowbufs.at[0, pl.ds(0, 16)],
                        agg_out.at[pl.ds(cid * N + NS * ROWS_T, 16)])
    _tile_node_out(tsp, t_out, sid, cid * N, zbuf)


def _tca_body(h_ref, deg_ref, w_ref, r_ref, hwr_ref):
    d = jnp.maximum(deg_ref[...], 1.0)
    r = lax.rsqrt(d)
    r_ref[...] = r
    for g in range(2):
        hw = jnp.dot(h_ref[g], w_ref[g], preferred_element_type=jnp.float32,
                     precision=lax.Precision.HIGHEST)
        hwr_ref[g] = r[g][:, None] * hw


_tca = pl.pallas_call(
    _tca_body,
    out_shape=(jax.ShapeDtypeStruct((2, N), jnp.float32),
               jax.ShapeDtypeStruct((2, N, D), jnp.float32)),
    compiler_params=pltpu.CompilerParams(vmem_limit_bytes=100 * 1024 * 1024),
)


def _tcb_body(agg_ref, r_ref, t_ref, w_ref, out_ref):
    for g in range(2):
        r = r_ref[g]
        x = r[:, None] * agg_ref[g]
        h1 = jnp.where(x >= 0, x, 0.01 * x)
        c = r * t_ref[g]
        s = jnp.dot(c[None, :], h1, preferred_element_type=jnp.float32,
                    precision=lax.Precision.HIGHEST)
        out_ref[g] = (jnp.dot(s, w_ref[g], preferred_element_type=jnp.float32,
                              precision=lax.Precision.HIGHEST) / float(N))[0]


_tcb = pl.pallas_call(
    _tcb_body,
    out_shape=jax.ShapeDtypeStruct((2, D), jnp.float32),
    compiler_params=pltpu.CompilerParams(vmem_limit_bytes=100 * 1024 * 1024),
)


def kernel(gu_ids, gu_edge_index, gv_ids, gv_edge_index, emb, Wp1, Wp2, Wc1, Wc2):
    ids2 = jnp.concatenate([gu_ids, gv_ids]).astype(jnp.int32)
    src2 = jnp.concatenate([gu_edge_index[0], gv_edge_index[0]]).astype(jnp.int32)
    dst2 = jnp.concatenate([gu_edge_index[1], gv_edge_index[1]]).astype(jnp.int32)
    emb = emb.astype(jnp.float32)

    h2, deg2 = _sc_deg_gather(ids2, dst2, emb)
    r, hwr = _tca(h2.reshape(2, N, D), deg2.reshape(2, N),
                  jnp.stack([Wp1, Wc1]))
    r_pad = jnp.pad(r.reshape(2 * N), (0, RPAD - 2 * N))
    agg2, t2 = _sc_segsum(src2, dst2, hwr.reshape(2 * N, D), r_pad)
    out = _tcb(agg2.reshape(2, N, D), r, t2.reshape(2, N),
               jnp.stack([Wp2, Wc2]))
    return out
